# Initial kernel scaffold; baseline (speedup 1.0000x reference)
import jax
import jax.numpy as jnp
from jax.experimental import pallas as pl


def kernel(x, adj_t, W0, b0, W1, b1, W2, b2, g1, beta1, g2, beta2):
    raise NotImplementedError("write your pallas kernel here")


# speedup vs baseline: 1.0366x; 1.0366x over previous
import functools

import jax
import jax.numpy as jnp
from jax.experimental import pallas as pl
from jax.experimental.pallas import tpu as pltpu

_BM = 200
_BM_PROJ = 1000


def _proj_body(x_ref, w_ref, o_ref):
    o_ref[...] = jnp.dot(x_ref[...], w_ref[...],
                         preferred_element_type=jnp.float32)


def _layer_body(adj_ref, p_ref, b_ref, g_ref, beta_ref, w_ref, o_ref):
    acc = jnp.dot(adj_ref[...], p_ref[...],
                  preferred_element_type=jnp.float32)
    h = acc + b_ref[...]
    mu = jnp.mean(h, axis=-1, keepdims=True)
    var = jnp.mean((h - mu) ** 2, axis=-1, keepdims=True)
    hn = (h - mu) / jnp.sqrt(var + 1e-5) * g_ref[...] + beta_ref[...]
    hr = jnp.maximum(hn, 0.0)
    o_ref[...] = jnp.dot(hr, w_ref[...], preferred_element_type=jnp.float32)


def _final_body(adj_ref, p_ref, b_ref, o_ref):
    acc = jnp.dot(adj_ref[...], p_ref[...],
                  preferred_element_type=jnp.float32)
    h = acc + b_ref[...]
    m = jnp.max(h, axis=-1, keepdims=True)
    e = jnp.exp(h - m)
    lse = jnp.log(jnp.sum(e, axis=-1, keepdims=True)) + m
    o_ref[...] = h - lse


def _proj(x, w):
    n, d = x.shape
    return pl.pallas_call(
        _proj_body,
        grid=(n // _BM_PROJ,),
        in_specs=[
            pl.BlockSpec((_BM_PROJ, d), lambda i: (i, 0)),
            pl.BlockSpec((d, w.shape[1]), lambda i: (0, 0)),
        ],
        out_specs=pl.BlockSpec((_BM_PROJ, w.shape[1]), lambda i: (i, 0)),
        out_shape=jax.ShapeDtypeStruct((n, w.shape[1]), jnp.float32),
    )(x, w)


def _layer(adj, p, b, g, beta, w_next):
    n, d = p.shape
    return pl.pallas_call(
        _layer_body,
        grid=(n // _BM,),
        in_specs=[
            pl.BlockSpec((_BM, n), lambda i: (i, 0)),
            pl.BlockSpec((n, d), lambda i: (0, 0)),
            pl.BlockSpec((1, d), lambda i: (0, 0)),
            pl.BlockSpec((1, d), lambda i: (0, 0)),
            pl.BlockSpec((1, d), lambda i: (0, 0)),
            pl.BlockSpec((d, d), lambda i: (0, 0)),
        ],
        out_specs=pl.BlockSpec((_BM, d), lambda i: (i, 0)),
        out_shape=jax.ShapeDtypeStruct((n, d), jnp.float32),
    )(adj, p, b, g, beta, w_next)


def _final(adj, p, b):
    n, d = p.shape
    return pl.pallas_call(
        _final_body,
        grid=(n // _BM,),
        in_specs=[
            pl.BlockSpec((_BM, n), lambda i: (i, 0)),
            pl.BlockSpec((n, d), lambda i: (0, 0)),
            pl.BlockSpec((1, d), lambda i: (0, 0)),
        ],
        out_specs=pl.BlockSpec((_BM, d), lambda i: (i, 0)),
        out_shape=jax.ShapeDtypeStruct((n, d), jnp.float32),
    )(adj, p, b)


def kernel(x, adj_t, W0, b0, W1, b1, W2, b2, g1, beta1, g2, beta2):
    r = lambda v: v.reshape(1, -1)
    p0 = _proj(x, W0)
    p1 = _layer(adj_t, p0, r(b0), r(g1), r(beta1), W1)
    p2 = _layer(adj_t, p1, r(b1), r(g2), r(beta2), W2)
    return _final(adj_t, p2, r(b2))


# speedup vs baseline: 1.1472x; 1.1066x over previous
import functools

import jax
import jax.numpy as jnp
from jax.experimental import pallas as pl
from jax.experimental.pallas import tpu as pltpu

_BM = 400
_BM_PROJ = 1000


def _proj_body(x_ref, w_ref, o_ref):
    o_ref[...] = jnp.dot(x_ref[...], w_ref[...],
                         preferred_element_type=jnp.float32)


def _layer_body(adj_ref, p_ref, b_ref, g_ref, beta_ref, w_ref, o_ref):
    acc = jnp.dot(adj_ref[...], p_ref[...],
                  preferred_element_type=jnp.float32)
    h = acc + b_ref[...]
    mu = jnp.mean(h, axis=-1, keepdims=True)
    var = jnp.mean((h - mu) ** 2, axis=-1, keepdims=True)
    hn = (h - mu) / jnp.sqrt(var + 1e-5) * g_ref[...] + beta_ref[...]
    hr = jnp.maximum(hn, 0.0)
    o_ref[...] = jnp.dot(hr, w_ref[...], preferred_element_type=jnp.float32)


def _final_body(adj_ref, p_ref, b_ref, o_ref):
    acc = jnp.dot(adj_ref[...], p_ref[...],
                  preferred_element_type=jnp.float32)
    h = acc + b_ref[...]
    m = jnp.max(h, axis=-1, keepdims=True)
    e = jnp.exp(h - m)
    lse = jnp.log(jnp.sum(e, axis=-1, keepdims=True)) + m
    o_ref[...] = h - lse


def _proj(x, w):
    n, d = x.shape
    return pl.pallas_call(
        _proj_body,
        grid=(n // _BM_PROJ,),
        in_specs=[
            pl.BlockSpec((_BM_PROJ, d), lambda i: (i, 0)),
            pl.BlockSpec((d, w.shape[1]), lambda i: (0, 0)),
        ],
        out_specs=pl.BlockSpec((_BM_PROJ, w.shape[1]), lambda i: (i, 0)),
        out_shape=jax.ShapeDtypeStruct((n, w.shape[1]), jnp.float32),
    )(x, w)


def _layer(adj, p, b, g, beta, w_next):
    n, d = p.shape
    return pl.pallas_call(
        _layer_body,
        grid=(n // _BM,),
        in_specs=[
            pl.BlockSpec((_BM, n), lambda i: (i, 0)),
            pl.BlockSpec((n, d), lambda i: (0, 0)),
            pl.BlockSpec((1, d), lambda i: (0, 0)),
            pl.BlockSpec((1, d), lambda i: (0, 0)),
            pl.BlockSpec((1, d), lambda i: (0, 0)),
            pl.BlockSpec((d, d), lambda i: (0, 0)),
        ],
        out_specs=pl.BlockSpec((_BM, d), lambda i: (i, 0)),
        out_shape=jax.ShapeDtypeStruct((n, d), jnp.float32),
    )(adj, p, b, g, beta, w_next)


def _final(adj, p, b):
    n, d = p.shape
    return pl.pallas_call(
        _final_body,
        grid=(n // _BM,),
        in_specs=[
            pl.BlockSpec((_BM, n), lambda i: (i, 0)),
            pl.BlockSpec((n, d), lambda i: (0, 0)),
            pl.BlockSpec((1, d), lambda i: (0, 0)),
        ],
        out_specs=pl.BlockSpec((_BM, d), lambda i: (i, 0)),
        out_shape=jax.ShapeDtypeStruct((n, d), jnp.float32),
    )(adj, p, b)


def kernel(x, adj_t, W0, b0, W1, b1, W2, b2, g1, beta1, g2, beta2):
    r = lambda v: v.reshape(1, -1)
    p0 = _proj(x, W0)
    p1 = _layer(adj_t, p0, r(b0), r(g1), r(beta1), W1)
    p2 = _layer(adj_t, p1, r(b1), r(g2), r(beta2), W2)
    return _final(adj_t, p2, r(b2))
